# baseline (speedup 1.0000x reference)
import functools

import jax
import jax.numpy as jnp
from jax import lax
from jax.experimental import pallas as pl
from jax.experimental.pallas import tpu as pltpu
from jax.experimental.pallas import tpu_sc as plsc

D_MODEL = 128
MAX_REL = 32
LENGTH = 1024
VOCAB = 2 * MAX_REL + 1
TROWS = 2048

_NUM_CORES = 2
_NUM_SUBCORES = 16
_NUM_WORKERS = _NUM_CORES * _NUM_SUBCORES

_SC_ROWS = 512
_TC_ROWS = LENGTH - _SC_ROWS
_SC_ROWS_PER_WORKER = _SC_ROWS // _NUM_WORKERS
_TC_BLOCK_ROWS = 8

_TMPL_CHUNK = TROWS // _NUM_SUBCORES
_LANES = 16


def _sc_fanout_body(tab_hbm, out_hbm, tmpl_sh, tab_v, chunk_v):
    c = lax.axis_index("c")
    s = lax.axis_index("s")

    pltpu.sync_copy(tab_hbm, tab_v)
    base_row = s * _TMPL_CHUNK

    def build(j, carry):
        g = base_row + j
        src = jnp.clip(g - (LENGTH - 1), -MAX_REL, MAX_REL) + MAX_REL
        for k in range(D_MODEL // _LANES):
            chunk_v[j, pl.ds(k * _LANES, _LANES)] = (
                tab_v[src, pl.ds(k * _LANES, _LANES)])
        return carry

    lax.fori_loop(0, _TMPL_CHUNK, build, 0)

    pltpu.sync_copy(chunk_v, tmpl_sh.at[pl.ds(base_row, _TMPL_CHUNK)])
    plsc.subcore_barrier()

    wid = s * _NUM_CORES + c
    base = wid * _SC_ROWS_PER_WORKER

    def row(r, carry):
        i = base + r
        start = (LENGTH - 1) - i
        pltpu.sync_copy(tmpl_sh.at[pl.ds(start, LENGTH)], out_hbm.at[i])
        return carry

    lax.fori_loop(0, _SC_ROWS_PER_WORKER, row, 0)


@functools.cache
def _sc_fanout():
    return pl.kernel(
        _sc_fanout_body,
        out_type=jax.ShapeDtypeStruct((LENGTH, LENGTH, D_MODEL), jnp.float32),
        mesh=plsc.VectorSubcoreMesh(core_axis_name="c", subcore_axis_name="s",
                                    num_cores=_NUM_CORES,
                                    num_subcores=_NUM_SUBCORES),
        scratch_types=[
            pltpu.VMEM_SHARED((TROWS, D_MODEL), jnp.float32),
            pltpu.VMEM((VOCAB, D_MODEL), jnp.float32),
            pltpu.VMEM((_TMPL_CHUNK, D_MODEL), jnp.float32),
        ],
    )


def _tc_fill_body(tab_ref, partial_ref, out_ref, tmpl_ref):
    del partial_ref
    pid = pl.program_id(0)

    @pl.when(pid == 0)
    def _():
        k = lax.broadcasted_iota(jnp.int32, (TROWS, 128), 0)
        v = lax.broadcasted_iota(jnp.int32, (TROWS, 128), 1)
        idx = jnp.clip(k - (LENGTH - 1), -MAX_REL, MAX_REL) + MAX_REL
        onehot = (idx == v).astype(jnp.float32)
        tmpl_ref[...] = lax.dot_general(
            onehot, tab_ref[...],
            dimension_numbers=(((1,), (0,)), ((), ())),
            preferred_element_type=jnp.float32,
        )

    for k in range(_TC_BLOCK_ROWS):
        i = _SC_ROWS + pid * _TC_BLOCK_ROWS + k
        start = (LENGTH - 1) - i
        pltpu.sync_copy(tmpl_ref.at[pl.ds(start, LENGTH), :], out_ref.at[k])


def _tc_fill(tab_padded, partial):
    return pl.pallas_call(
        _tc_fill_body,
        grid=(_TC_ROWS // _TC_BLOCK_ROWS,),
        in_specs=[
            pl.BlockSpec((128, D_MODEL), lambda i: (0, 0)),
            pl.BlockSpec(memory_space=pl.ANY),
        ],
        out_specs=pl.BlockSpec((_TC_BLOCK_ROWS, LENGTH, D_MODEL),
                               lambda i: (i + _SC_ROWS // _TC_BLOCK_ROWS, 0, 0)),
        out_shape=jax.ShapeDtypeStruct((LENGTH, LENGTH, D_MODEL), jnp.float32),
        scratch_shapes=[pltpu.VMEM((TROWS, D_MODEL), jnp.float32)],
        input_output_aliases={1: 0},
    )(tab_padded, partial)


def kernel(length, rel_pos_embeddings):
    del length
    tab = rel_pos_embeddings.astype(jnp.float32)
    tab_padded = jnp.zeros((128, D_MODEL), jnp.float32)
    tab_padded = lax.dynamic_update_slice(tab_padded, tab, (0, 0))
    partial = _sc_fanout()(tab)
    return _tc_fill(tab_padded, partial)
